# baseline (speedup 1.0000x reference)
import functools

import jax
import jax.numpy as jnp
from jax.experimental import pallas as pl
from jax.experimental.pallas import tpu as pltpu

_BB = 32


def _tab_kernel(ce_ref, pos_ref, gam_ref, bet_ref, tab_ref):
    ce = ce_ref[...]
    pos = pos_ref[...]
    x = ce[:, None, :] + pos[None, :, :]
    mu = jnp.mean(x, axis=-1, keepdims=True)
    xc = x - mu
    var = jnp.mean(xc * xc, axis=-1, keepdims=True)
    xn = xc * jax.lax.rsqrt(var + 1e-5)
    tab_ref[...] = xn * gam_ref[0][None, None, :] + bet_ref[0][None, None, :]


def _select_kernel(g_ref, tab_ref, out_ref, *, num_colors):
    g = g_ref[...]
    gexp = jnp.repeat(g, 64, axis=1)
    tab = tab_ref[...]
    acc = jnp.broadcast_to(tab[0:1, :], gexp.shape)
    for c in range(1, num_colors):
        acc = jnp.where(gexp == c, jnp.broadcast_to(tab[c:c + 1, :], gexp.shape), acc)
    out_ref[...] = acc


def kernel(grid, color_embed, pos_embed, gamma, beta):
    B, H, W = grid.shape
    P = H * W
    C, D = color_embed.shape

    posf = pos_embed[0, :H, :W, :].reshape(P, D)

    tab = pl.pallas_call(
        _tab_kernel,
        out_shape=jax.ShapeDtypeStruct((C, P, D), jnp.float32),
    )(color_embed, posf, gamma.reshape(1, D), beta.reshape(1, D))

    g2 = grid.reshape(B, P)
    tabf = tab.reshape(C, P * D)

    out = pl.pallas_call(
        functools.partial(_select_kernel, num_colors=C),
        grid=(B // _BB,),
        in_specs=[
            pl.BlockSpec((_BB, P), lambda i: (i, 0)),
            pl.BlockSpec((C, P * D), lambda i: (0, 0)),
        ],
        out_specs=pl.BlockSpec((_BB, P * D), lambda i: (i, 0)),
        out_shape=jax.ShapeDtypeStruct((B, P * D), jnp.float32),
    )(g2, tabf)
    return out.reshape(B, P, D)
